# baseline (speedup 1.0000x reference)
import jax
import jax.numpy as jnp
from jax import lax
from jax.experimental import pallas as pl
from jax.experimental.pallas import tpu as pltpu
from jax.experimental.pallas import tpu_sc as plsc

_BATCH = 16384
_FEAT = 64
_NC = 2
_NS = 16
_NW = _NC * _NS
_BPW = _BATCH // _NW
_CHUNK = 128
_NCHUNK = _BPW // _CHUNK
_L = 16


def _center_loss_body(feat_hbm, lab_hbm, cent_hbm, out_hbm,
                      idx_v, rows_v, feat_v, part_v, sem):
    cid = lax.axis_index("c")
    sid = lax.axis_index("s")
    wid = sid * _NC + cid

    pltpu.sync_copy(lab_hbm.at[wid], idx_v)
    gathers = [
        pltpu.async_copy(cent_hbm.at[idx_v.at[j]], rows_v.at[j], sem)
        for j in range(_NCHUNK)
    ]
    pltpu.sync_copy(feat_hbm.at[wid], feat_v)
    for g in gathers:
        g.wait()

    zero = jnp.zeros((_L,), jnp.float32)
    accs = (zero, zero, zero, zero)
    for j in range(_NCHUNK):
        def body(r, accs, j=j):
            out = []
            for c in range(_FEAT // _L):
                f = feat_v[j, r, pl.ds(c * _L, _L)]
                g = rows_v[j, r, pl.ds(c * _L, _L)]
                d = f - g
                out.append(accs[c] + d * d)
            return tuple(out)
        accs = lax.fori_loop(0, _CHUNK, body, accs)

    acc = (accs[0] + accs[1]) + (accs[2] + accs[3])
    part_v[...] = acc * (1.0 / _BATCH)
    pltpu.sync_copy(part_v, out_hbm.at[wid])


@jax.jit
def _center_loss(features, labels, centers):
    mesh = plsc.VectorSubcoreMesh(core_axis_name="c", subcore_axis_name="s")
    kfn = pl.kernel(
        _center_loss_body,
        mesh=mesh,
        compiler_params=pltpu.CompilerParams(use_tc_tiling_on_sc=False),
        out_type=jax.ShapeDtypeStruct((_NW, _L), jnp.float32),
        scratch_types=[
            pltpu.VMEM((_NCHUNK, _CHUNK), jnp.int32),
            pltpu.VMEM((_NCHUNK, _CHUNK, _FEAT), jnp.float32),
            pltpu.VMEM((_NCHUNK, _CHUNK, _FEAT), jnp.float32),
            pltpu.VMEM((_L,), jnp.float32),
            pltpu.SemaphoreType.DMA,
        ],
    )
    feats = features.reshape(_NW, _NCHUNK, _CHUNK, _FEAT)
    labs = labels.astype(jnp.int32).reshape(_NW, _NCHUNK, _CHUNK)
    out = kfn(feats, labs, centers)
    return jnp.sum(out)


def kernel(features, labels, centers):
    return _center_loss(features, labels, centers)
